# baseline (speedup 1.0000x reference)
import functools
import math

import jax
import jax.numpy as jnp
from jax import lax
from jax.experimental import pallas as pl
from jax.experimental.pallas import tpu as pltpu
from jax.experimental.pallas import tpu_sc as plsc

N = 10000
E = 320000
D = 128
EPS = 1e-5
BN_SCALE = 1.0 / math.sqrt(1.0 + EPS)

NC = 2
NS = 16
NW = NC * NS
EPW = E // NW
CB = 40
NCHUNK = EPW // CB
NBUF = 6
RPS = 1000

_mesh = plsc.VectorSubcoreMesh(core_axis_name="c", subcore_axis_name="s")


@functools.partial(
    pl.kernel,
    out_type=jax.ShapeDtypeStruct((NW, 1, N), jnp.float32),
    mesh=_mesh,
    compiler_params=pltpu.CompilerParams(needs_layout_passes=False),
    scratch_types=[
        pltpu.VMEM((EPW,), jnp.int32),
        pltpu.VMEM((N,), jnp.float32),
        pltpu.SemaphoreType.DMA,
    ],
)
def _sc_degree(col_hbm, out_hbm, idx_v, deg_v, sem):
    cid = lax.axis_index("c")
    sid = lax.axis_index("s")
    wid = sid * NC + cid

    def _z(i, _):
        deg_v[pl.ds(i * 16, 16)] = jnp.zeros((16,), jnp.float32)
        return 0

    lax.fori_loop(0, N // 16, _z, 0)
    pltpu.sync_copy(col_hbm.at[pl.ds(wid * EPW, EPW)], idx_v)
    ones = jnp.ones((16,), jnp.float32)

    def _e(j, _):
        idx = idx_v[pl.ds(j * 16, 16)]
        plsc.addupdate_scatter(deg_v, [idx], ones)
        return 0

    lax.fori_loop(0, EPW // 16, _e, 0)
    pltpu.sync_copy(deg_v, out_hbm.at[wid, 0])


NGRP = (NCHUNK + NBUF - 1) // NBUF


@functools.partial(
    pl.kernel,
    out_type=jax.ShapeDtypeStruct((NC, N, D), jnp.float32),
    mesh=_mesh,
    scratch_types=[
        pltpu.VMEM((EPW,), jnp.int32),
        pltpu.VMEM((EPW,), jnp.int32),
    ] + [pltpu.VMEM((CB, D), jnp.float32) for _ in range(NBUF)] + [
        pltpu.VMEM_SHARED((N, D), jnp.float32),
        pltpu.SemaphoreType.DMA,
        pltpu.SemaphoreType.DMA,
        pltpu.SemaphoreType.DMA,
    ],
)
def _sc_scatter(row_hbm, col_hbm, hs_hbm, znd_hbm, out_hbm, *rest):
    idxr_v, idxc_v = rest[0], rest[1]
    gats = list(rest[2:2 + NBUF])
    acc_s = rest[2 + NBUF]
    gsem, psem, zsem = rest[3 + NBUF], rest[4 + NBUF], rest[5 + NBUF]
    cid = lax.axis_index("c")
    sid = lax.axis_index("s")
    wid = sid * NC + cid

    pltpu.async_copy(row_hbm.at[pl.ds(wid * EPW, EPW)], idxr_v, psem)
    pltpu.async_copy(col_hbm.at[pl.ds(wid * EPW, EPW)], idxc_v, zsem)

    @pl.when(sid < 10)
    def _zero():
        pltpu.async_copy(znd_hbm.at[pl.ds(sid * RPS, RPS)],
                         acc_s.at[pl.ds(sid * RPS, RPS)], zsem)

    def _r(k):
        return idxr_v.at[pl.ds(k * CB, CB)]

    def _c(k):
        return idxc_v.at[pl.ds(k * CB, CB)]

    pltpu.make_async_copy(row_hbm.at[pl.ds(wid * EPW, EPW)], idxr_v,
                          psem).wait()

    for j in range(NBUF - 1):
        pltpu.async_copy(hs_hbm.at[_r(j)], gats[j], gsem)

    pltpu.make_async_copy(col_hbm.at[pl.ds(wid * EPW, EPW)], idxc_v,
                          zsem).wait()

    @pl.when(sid < 10)
    def _zwait():
        pltpu.make_async_copy(znd_hbm.at[pl.ds(sid * RPS, RPS)],
                              acc_s.at[pl.ds(sid * RPS, RPS)], zsem).wait()

    plsc.subcore_barrier()

    def _grp(g, _):
        for j in range(NBUF):
            k = g * NBUF + j

            @pl.when(k < NCHUNK)
            def _body():
                pltpu.make_async_copy(hs_hbm.at[_r(k)], gats[j],
                                      gsem).wait()

                @pl.when(k + NBUF - 1 < NCHUNK)
                def _nxt():
                    pltpu.async_copy(hs_hbm.at[_r(k + NBUF - 1)],
                                     gats[(j - 1) % NBUF], gsem)

                pltpu.sync_copy(gats[j], acc_s.at[_c(k)], add=True)
        return 0

    lax.fori_loop(0, NGRP, _grp, 0)

    plsc.subcore_barrier()

    @pl.when(sid < 10)
    def _drain():
        pltpu.sync_copy(acc_s.at[pl.ds(sid * RPS, RPS)],
                        out_hbm.at[cid, pl.ds(sid * RPS, RPS)])


BT = 2000


def _dinv_body(degp_ref, o_ref):
    deg = jnp.sum(degp_ref[...], axis=0)
    o_ref[...] = jnp.where(deg > 0.0,
                           lax.rsqrt(jnp.maximum(deg, 1e-12)), 0.0)


_dinv = pl.pallas_call(
    _dinv_body,
    out_shape=jax.ShapeDtypeStruct((1, N), jnp.float32),
)


def _mm_body(x_ref, w_ref, o_ref):
    o_ref[...] = jnp.dot(x_ref[...], w_ref[...],
                         preferred_element_type=jnp.float32)


_mm = pl.pallas_call(
    _mm_body,
    grid=(N // BT,),
    in_specs=[
        pl.BlockSpec((BT, D), lambda i: (i, 0)),
        pl.BlockSpec((D, D), lambda i: (0, 0)),
    ],
    out_specs=pl.BlockSpec((BT, D), lambda i: (i, 0)),
    out_shape=jax.ShapeDtypeStruct((N, D), jnp.float32),
)


def _scale_body(h_ref, dinv_ref, o_ref):
    o_ref[...] = h_ref[...] * dinv_ref[...]


_scale = pl.pallas_call(
    _scale_body,
    grid=(N // BT,),
    in_specs=[
        pl.BlockSpec((BT, D), lambda i: (i, 0)),
        pl.BlockSpec((BT, 1), lambda i: (i, 0)),
    ],
    out_specs=pl.BlockSpec((BT, D), lambda i: (i, 0)),
    out_shape=jax.ShapeDtypeStruct((N, D), jnp.float32),
)


def _epi_res_body(aggp_ref, dinv_ref, b_ref, g_ref, be_ref, w_ref, xp_ref,
                  x_ref, hs_ref):
    dinv = dinv_ref[...]
    agg = (aggp_ref[0] + aggp_ref[1]) * dinv
    x = (agg + b_ref[...]) * (BN_SCALE * g_ref[...]) + be_ref[...]
    x = jnp.maximum(x, 0.0) + 0.5 * xp_ref[...]
    x_ref[...] = x
    hs_ref[...] = jnp.dot(x, w_ref[...],
                          preferred_element_type=jnp.float32) * dinv


def _epi0_body(aggp_ref, dinv_ref, b_ref, g_ref, be_ref, w_ref,
               x_ref, hs_ref):
    dinv = dinv_ref[...]
    agg = (aggp_ref[0] + aggp_ref[1]) * dinv
    x = (agg + b_ref[...]) * (BN_SCALE * g_ref[...]) + be_ref[...]
    x = jnp.maximum(x, 0.0)
    x_ref[...] = x
    hs_ref[...] = jnp.dot(x, w_ref[...],
                          preferred_element_type=jnp.float32) * dinv


def _fin_body(aggp_ref, dinv_ref, b_ref, g_ref, be_ref, xp_ref, x_ref):
    agg = (aggp_ref[0] + aggp_ref[1]) * dinv_ref[...]
    x = (agg + b_ref[...]) * (BN_SCALE * g_ref[...]) + be_ref[...]
    x_ref[...] = jnp.maximum(x, 0.0) + 0.5 * xp_ref[...]


_spec_aggp = pl.BlockSpec((2, BT, D), lambda i: (0, i, 0))
_spec_rows = pl.BlockSpec((BT, D), lambda i: (i, 0))
_spec_dinv = pl.BlockSpec((BT, 1), lambda i: (i, 0))
_spec_vec = pl.BlockSpec((D,), lambda i: (0,))
_spec_w = pl.BlockSpec((D, D), lambda i: (0, 0))

_epi0 = pl.pallas_call(
    _epi0_body,
    grid=(N // BT,),
    in_specs=[_spec_aggp, _spec_dinv, _spec_vec, _spec_vec, _spec_vec,
              _spec_w],
    out_specs=(_spec_rows, _spec_rows),
    out_shape=(jax.ShapeDtypeStruct((N, D), jnp.float32),
               jax.ShapeDtypeStruct((N, D), jnp.float32)),
)

_epi_res = pl.pallas_call(
    _epi_res_body,
    grid=(N // BT,),
    in_specs=[_spec_aggp, _spec_dinv, _spec_vec, _spec_vec, _spec_vec,
              _spec_w, _spec_rows],
    out_specs=(_spec_rows, _spec_rows),
    out_shape=(jax.ShapeDtypeStruct((N, D), jnp.float32),
               jax.ShapeDtypeStruct((N, D), jnp.float32)),
)

_fin = pl.pallas_call(
    _fin_body,
    grid=(N // BT,),
    in_specs=[_spec_aggp, _spec_dinv, _spec_vec, _spec_vec, _spec_vec,
              _spec_rows],
    out_specs=_spec_rows,
    out_shape=jax.ShapeDtypeStruct((N, D), jnp.float32),
)


def kernel(emb, W0, b0, gamma0, beta0, W1, b1, gamma1, beta1,
           W2, b2, gamma2, beta2, edge_index):
    row = edge_index[0]
    col = edge_index[1]
    znd = jnp.zeros((N, D), jnp.float32)

    h0 = _mm(emb, W0)
    degp = _sc_degree(col)
    dinv = jnp.reshape(_dinv(degp), (N, 1))

    hs = _scale(h0, dinv)
    aggp = _sc_scatter(row, col, hs, znd)
    x1, hs = _epi0(aggp, dinv, b0, gamma0, beta0, W1)
    aggp = _sc_scatter(row, col, hs, znd)
    x2, hs = _epi_res(aggp, dinv, b1, gamma1, beta1, W2, x1)
    aggp = _sc_scatter(row, col, hs, znd)
    return _fin(aggp, dinv, b2, gamma2, beta2, x2)


# speedup vs baseline: 1.0002x; 1.0002x over previous
import functools
import math

import jax
import jax.numpy as jnp
from jax import lax
from jax.experimental import pallas as pl
from jax.experimental.pallas import tpu as pltpu
from jax.experimental.pallas import tpu_sc as plsc

N = 10000
E = 320000
D = 128
EPS = 1e-5
BN_SCALE = 1.0 / math.sqrt(1.0 + EPS)

NC = 2
NS = 16
NW = NC * NS
EPW = E // NW
CB = 40
NCHUNK = EPW // CB
NBUF = 6
RPS = 1000

_mesh = plsc.VectorSubcoreMesh(core_axis_name="c", subcore_axis_name="s")


@functools.partial(
    pl.kernel,
    out_type=jax.ShapeDtypeStruct((NW, 1, N), jnp.float32),
    mesh=_mesh,
    compiler_params=pltpu.CompilerParams(needs_layout_passes=False),
    scratch_types=[
        pltpu.VMEM((EPW,), jnp.int32),
        pltpu.VMEM((N,), jnp.float32),
        pltpu.SemaphoreType.DMA,
    ],
)
def _sc_degree(col_hbm, out_hbm, idx_v, deg_v, sem):
    cid = lax.axis_index("c")
    sid = lax.axis_index("s")
    wid = sid * NC + cid

    def _z(i, _):
        deg_v[pl.ds(i * 16, 16)] = jnp.zeros((16,), jnp.float32)
        return 0

    lax.fori_loop(0, N // 16, _z, 0)
    pltpu.sync_copy(col_hbm.at[pl.ds(wid * EPW, EPW)], idx_v)
    ones = jnp.ones((16,), jnp.float32)

    def _e(j, _):
        idx = idx_v[pl.ds(j * 16, 16)]
        plsc.addupdate_scatter(deg_v, [idx], ones)
        return 0

    lax.fori_loop(0, EPW // 16, _e, 0)
    pltpu.sync_copy(deg_v, out_hbm.at[wid, 0])


NGRP = (NCHUNK + NBUF - 1) // NBUF


@functools.partial(
    pl.kernel,
    out_type=jax.ShapeDtypeStruct((NC, N, D), jnp.float32),
    mesh=_mesh,
    scratch_types=[
        pltpu.VMEM((EPW,), jnp.int32),
        pltpu.VMEM((EPW,), jnp.int32),
    ] + [pltpu.VMEM((CB, D), jnp.float32) for _ in range(NBUF)] + [
        pltpu.VMEM_SHARED((N, D), jnp.float32),
        pltpu.SemaphoreType.DMA,
        pltpu.SemaphoreType.DMA,
        pltpu.SemaphoreType.DMA,
    ],
)
def _sc_scatter(row_hbm, col_hbm, hs_hbm, znd_hbm, out_hbm, *rest):
    idxr_v, idxc_v = rest[0], rest[1]
    gats = list(rest[2:2 + NBUF])
    acc_s = rest[2 + NBUF]
    gsem, psem, zsem = rest[3 + NBUF], rest[4 + NBUF], rest[5 + NBUF]
    cid = lax.axis_index("c")
    sid = lax.axis_index("s")
    wid = sid * NC + cid

    pltpu.async_copy(row_hbm.at[pl.ds(wid * EPW, EPW)], idxr_v, psem)
    pltpu.async_copy(col_hbm.at[pl.ds(wid * EPW, EPW)], idxc_v, zsem)

    @pl.when(sid < 10)
    def _zero():
        pltpu.async_copy(znd_hbm.at[pl.ds(sid * RPS, RPS)],
                         acc_s.at[pl.ds(sid * RPS, RPS)], zsem)

    def _r(k):
        return idxr_v.at[pl.ds(k * CB, CB)]

    def _c(k):
        return idxc_v.at[pl.ds(k * CB, CB)]

    pltpu.make_async_copy(row_hbm.at[pl.ds(wid * EPW, EPW)], idxr_v,
                          psem).wait()

    for j in range(NBUF - 1):
        pltpu.async_copy(hs_hbm.at[_r(j)], gats[j], gsem)

    pltpu.make_async_copy(col_hbm.at[pl.ds(wid * EPW, EPW)], idxc_v,
                          zsem).wait()

    @pl.when(sid < 10)
    def _zwait():
        pltpu.make_async_copy(znd_hbm.at[pl.ds(sid * RPS, RPS)],
                              acc_s.at[pl.ds(sid * RPS, RPS)], zsem).wait()

    plsc.subcore_barrier()

    def _grp(g, _):
        for j in range(NBUF):
            k = g * NBUF + j

            @pl.when(k < NCHUNK)
            def _body():
                pltpu.make_async_copy(hs_hbm.at[_r(k)], gats[j],
                                      gsem).wait()

                @pl.when(k + NBUF - 1 < NCHUNK)
                def _nxt():
                    pltpu.async_copy(hs_hbm.at[_r(k + NBUF - 1)],
                                     gats[(j - 1) % NBUF], gsem)

                pltpu.sync_copy(gats[j], acc_s.at[_c(k)], add=True)
        return 0

    lax.fori_loop(0, NGRP, _grp, 0)

    plsc.subcore_barrier()

    @pl.when(sid < 10)
    def _drain():
        pltpu.sync_copy(acc_s.at[pl.ds(sid * RPS, RPS)],
                        out_hbm.at[cid, pl.ds(sid * RPS, RPS)])


BT = 2000


def _dinv_body(degp_ref, o_ref):
    deg = jnp.sum(degp_ref[...], axis=0)
    o_ref[...] = jnp.where(deg > 0.0,
                           lax.rsqrt(jnp.maximum(deg, 1e-12)), 0.0)


_dinv = pl.pallas_call(
    _dinv_body,
    out_shape=jax.ShapeDtypeStruct((1, N), jnp.float32),
)


def _mms_body(x_ref, w_ref, dinv_ref, o_ref):
    h = jnp.dot(x_ref[...], w_ref[...], preferred_element_type=jnp.float32)
    o_ref[...] = h * dinv_ref[...]


_mm_scale = pl.pallas_call(
    _mms_body,
    grid=(N // BT,),
    in_specs=[
        pl.BlockSpec((BT, D), lambda i: (i, 0)),
        pl.BlockSpec((D, D), lambda i: (0, 0)),
        pl.BlockSpec((BT, 1), lambda i: (i, 0)),
    ],
    out_specs=pl.BlockSpec((BT, D), lambda i: (i, 0)),
    out_shape=jax.ShapeDtypeStruct((N, D), jnp.float32),
)


def _epi_res_body(aggp_ref, dinv_ref, b_ref, g_ref, be_ref, w_ref, xp_ref,
                  x_ref, hs_ref):
    dinv = dinv_ref[...]
    agg = (aggp_ref[0] + aggp_ref[1]) * dinv
    x = (agg + b_ref[...]) * (BN_SCALE * g_ref[...]) + be_ref[...]
    x = jnp.maximum(x, 0.0) + 0.5 * xp_ref[...]
    x_ref[...] = x
    hs_ref[...] = jnp.dot(x, w_ref[...],
                          preferred_element_type=jnp.float32) * dinv


def _epi0_body(aggp_ref, dinv_ref, b_ref, g_ref, be_ref, w_ref,
               x_ref, hs_ref):
    dinv = dinv_ref[...]
    agg = (aggp_ref[0] + aggp_ref[1]) * dinv
    x = (agg + b_ref[...]) * (BN_SCALE * g_ref[...]) + be_ref[...]
    x = jnp.maximum(x, 0.0)
    x_ref[...] = x
    hs_ref[...] = jnp.dot(x, w_ref[...],
                          preferred_element_type=jnp.float32) * dinv


def _fin_body(aggp_ref, dinv_ref, b_ref, g_ref, be_ref, xp_ref, x_ref):
    agg = (aggp_ref[0] + aggp_ref[1]) * dinv_ref[...]
    x = (agg + b_ref[...]) * (BN_SCALE * g_ref[...]) + be_ref[...]
    x_ref[...] = jnp.maximum(x, 0.0) + 0.5 * xp_ref[...]


_spec_aggp = pl.BlockSpec((2, BT, D), lambda i: (0, i, 0))
_spec_rows = pl.BlockSpec((BT, D), lambda i: (i, 0))
_spec_dinv = pl.BlockSpec((BT, 1), lambda i: (i, 0))
_spec_vec = pl.BlockSpec((D,), lambda i: (0,))
_spec_w = pl.BlockSpec((D, D), lambda i: (0, 0))

_epi0 = pl.pallas_call(
    _epi0_body,
    grid=(N // BT,),
    in_specs=[_spec_aggp, _spec_dinv, _spec_vec, _spec_vec, _spec_vec,
              _spec_w],
    out_specs=(_spec_rows, _spec_rows),
    out_shape=(jax.ShapeDtypeStruct((N, D), jnp.float32),
               jax.ShapeDtypeStruct((N, D), jnp.float32)),
)

_epi_res = pl.pallas_call(
    _epi_res_body,
    grid=(N // BT,),
    in_specs=[_spec_aggp, _spec_dinv, _spec_vec, _spec_vec, _spec_vec,
              _spec_w, _spec_rows],
    out_specs=(_spec_rows, _spec_rows),
    out_shape=(jax.ShapeDtypeStruct((N, D), jnp.float32),
               jax.ShapeDtypeStruct((N, D), jnp.float32)),
)

_fin = pl.pallas_call(
    _fin_body,
    grid=(N // BT,),
    in_specs=[_spec_aggp, _spec_dinv, _spec_vec, _spec_vec, _spec_vec,
              _spec_rows],
    out_specs=_spec_rows,
    out_shape=jax.ShapeDtypeStruct((N, D), jnp.float32),
)


def kernel(emb, W0, b0, gamma0, beta0, W1, b1, gamma1, beta1,
           W2, b2, gamma2, beta2, edge_index):
    row = edge_index[0]
    col = edge_index[1]
    znd = jnp.zeros((N, D), jnp.float32)

    degp = _sc_degree(col)
    dinv = jnp.reshape(_dinv(degp), (N, 1))

    hs = _mm_scale(emb, W0, dinv)
    aggp = _sc_scatter(row, col, hs, znd)
    x1, hs = _epi0(aggp, dinv, b0, gamma0, beta0, W1)
    aggp = _sc_scatter(row, col, hs, znd)
    x2, hs = _epi_res(aggp, dinv, b1, gamma1, beta1, W2, x1)
    aggp = _sc_scatter(row, col, hs, znd)
    return _fin(aggp, dinv, b2, gamma2, beta2, x2)


# speedup vs baseline: 1.0007x; 1.0005x over previous
import functools
import math

import jax
import jax.numpy as jnp
from jax import lax
from jax.experimental import pallas as pl
from jax.experimental.pallas import tpu as pltpu
from jax.experimental.pallas import tpu_sc as plsc

N = 10000
E = 320000
D = 128
EPS = 1e-5
BN_SCALE = 1.0 / math.sqrt(1.0 + EPS)

NC = 2
NS = 16
NW = NC * NS
EPW = E // NW
CB = 40
NCHUNK = EPW // CB
NBUF = 6
RPS = 1000

_mesh = plsc.VectorSubcoreMesh(core_axis_name="c", subcore_axis_name="s")


@functools.partial(
    pl.kernel,
    out_type=jax.ShapeDtypeStruct((NW, 1, N), jnp.float32),
    mesh=_mesh,
    compiler_params=pltpu.CompilerParams(needs_layout_passes=False),
    scratch_types=[
        pltpu.VMEM((EPW,), jnp.int32),
        pltpu.VMEM((N,), jnp.float32),
        pltpu.SemaphoreType.DMA,
        pltpu.SemaphoreType.DMA,
    ],
)
def _sc_degree(col_hbm, z1_hbm, out_hbm, idx_v, deg_v, isem, zsem):
    cid = lax.axis_index("c")
    sid = lax.axis_index("s")
    wid = sid * NC + cid

    pltpu.async_copy(col_hbm.at[pl.ds(wid * EPW, EPW)], idx_v, isem)
    pltpu.async_copy(z1_hbm, deg_v, zsem)
    pltpu.make_async_copy(z1_hbm, deg_v, zsem).wait()
    pltpu.make_async_copy(col_hbm.at[pl.ds(wid * EPW, EPW)], idx_v,
                          isem).wait()
    ones = jnp.ones((16,), jnp.float32)

    def _e(j, _):
        for u in range(5):
            idx = idx_v[pl.ds((5 * j + u) * 16, 16)]
            plsc.addupdate_scatter(deg_v, [idx], ones)
        return 0

    lax.fori_loop(0, EPW // 80, _e, 0)
    pltpu.sync_copy(deg_v, out_hbm.at[wid, 0])


NGRP = (NCHUNK + NBUF - 1) // NBUF


@functools.partial(
    pl.kernel,
    out_type=jax.ShapeDtypeStruct((NC, N, D), jnp.float32),
    mesh=_mesh,
    scratch_types=[
        pltpu.VMEM((EPW,), jnp.int32),
        pltpu.VMEM((EPW,), jnp.int32),
    ] + [pltpu.VMEM((CB, D), jnp.float32) for _ in range(NBUF)] + [
        pltpu.VMEM_SHARED((N, D), jnp.float32),
        pltpu.SemaphoreType.DMA,
        pltpu.SemaphoreType.DMA,
        pltpu.SemaphoreType.DMA,
    ],
)
def _sc_scatter(row_hbm, col_hbm, hs_hbm, znd_hbm, out_hbm, *rest):
    idxr_v, idxc_v = rest[0], rest[1]
    gats = list(rest[2:2 + NBUF])
    acc_s = rest[2 + NBUF]
    gsem, psem, zsem = rest[3 + NBUF], rest[4 + NBUF], rest[5 + NBUF]
    cid = lax.axis_index("c")
    sid = lax.axis_index("s")
    wid = sid * NC + cid

    pltpu.async_copy(row_hbm.at[pl.ds(wid * EPW, EPW)], idxr_v, psem)
    pltpu.async_copy(col_hbm.at[pl.ds(wid * EPW, EPW)], idxc_v, zsem)

    @pl.when(sid < 10)
    def _zero():
        pltpu.async_copy(znd_hbm.at[pl.ds(sid * RPS, RPS)],
                         acc_s.at[pl.ds(sid * RPS, RPS)], zsem)

    def _r(k):
        return idxr_v.at[pl.ds(k * CB, CB)]

    def _c(k):
        return idxc_v.at[pl.ds(k * CB, CB)]

    pltpu.make_async_copy(row_hbm.at[pl.ds(wid * EPW, EPW)], idxr_v,
                          psem).wait()

    for j in range(NBUF - 1):
        pltpu.async_copy(hs_hbm.at[_r(j)], gats[j], gsem)

    pltpu.make_async_copy(col_hbm.at[pl.ds(wid * EPW, EPW)], idxc_v,
                          zsem).wait()

    @pl.when(sid < 10)
    def _zwait():
        pltpu.make_async_copy(znd_hbm.at[pl.ds(sid * RPS, RPS)],
                              acc_s.at[pl.ds(sid * RPS, RPS)], zsem).wait()

    plsc.subcore_barrier()

    def _grp(g, _):
        for j in range(NBUF):
            k = g * NBUF + j

            @pl.when(k < NCHUNK)
            def _body():
                pltpu.make_async_copy(hs_hbm.at[_r(k)], gats[j],
                                      gsem).wait()

                @pl.when(k + NBUF - 1 < NCHUNK)
                def _nxt():
                    pltpu.async_copy(hs_hbm.at[_r(k + NBUF - 1)],
                                     gats[(j - 1) % NBUF], gsem)

                pltpu.sync_copy(gats[j], acc_s.at[_c(k)], add=True)
        return 0

    lax.fori_loop(0, NGRP, _grp, 0)

    plsc.subcore_barrier()

    @pl.when(sid < 10)
    def _drain():
        pltpu.sync_copy(acc_s.at[pl.ds(sid * RPS, RPS)],
                        out_hbm.at[cid, pl.ds(sid * RPS, RPS)])


BT = 2000


def _dinv_body(degp_ref, o_ref):
    deg = jnp.sum(degp_ref[...], axis=0)
    o_ref[...] = jnp.where(deg > 0.0,
                           lax.rsqrt(jnp.maximum(deg, 1e-12)), 0.0)


_dinv = pl.pallas_call(
    _dinv_body,
    out_shape=jax.ShapeDtypeStruct((1, N), jnp.float32),
)


def _mms_body(x_ref, w_ref, dinv_ref, o_ref):
    h = jnp.dot(x_ref[...], w_ref[...], preferred_element_type=jnp.float32)
    o_ref[...] = h * dinv_ref[...]


_mm_scale = pl.pallas_call(
    _mms_body,
    grid=(N // BT,),
    in_specs=[
        pl.BlockSpec((BT, D), lambda i: (i, 0)),
        pl.BlockSpec((D, D), lambda i: (0, 0)),
        pl.BlockSpec((BT, 1), lambda i: (i, 0)),
    ],
    out_specs=pl.BlockSpec((BT, D), lambda i: (i, 0)),
    out_shape=jax.ShapeDtypeStruct((N, D), jnp.float32),
)


def _epi_res_body(aggp_ref, dinv_ref, b_ref, g_ref, be_ref, w_ref, xp_ref,
                  x_ref, hs_ref):
    dinv = dinv_ref[...]
    agg = (aggp_ref[0] + aggp_ref[1]) * dinv
    x = (agg + b_ref[...]) * (BN_SCALE * g_ref[...]) + be_ref[...]
    x = jnp.maximum(x, 0.0) + 0.5 * xp_ref[...]
    x_ref[...] = x
    hs_ref[...] = jnp.dot(x, w_ref[...],
                          preferred_element_type=jnp.float32) * dinv


def _epi0_body(aggp_ref, dinv_ref, b_ref, g_ref, be_ref, w_ref,
               x_ref, hs_ref):
    dinv = dinv_ref[...]
    agg = (aggp_ref[0] + aggp_ref[1]) * dinv
    x = (agg + b_ref[...]) * (BN_SCALE * g_ref[...]) + be_ref[...]
    x = jnp.maximum(x, 0.0)
    x_ref[...] = x
    hs_ref[...] = jnp.dot(x, w_ref[...],
                          preferred_element_type=jnp.float32) * dinv


def _fin_body(aggp_ref, dinv_ref, b_ref, g_ref, be_ref, xp_ref, x_ref):
    agg = (aggp_ref[0] + aggp_ref[1]) * dinv_ref[...]
    x = (agg + b_ref[...]) * (BN_SCALE * g_ref[...]) + be_ref[...]
    x_ref[...] = jnp.maximum(x, 0.0) + 0.5 * xp_ref[...]


_spec_aggp = pl.BlockSpec((2, BT, D), lambda i: (0, i, 0))
_spec_rows = pl.BlockSpec((BT, D), lambda i: (i, 0))
_spec_dinv = pl.BlockSpec((BT, 1), lambda i: (i, 0))
_spec_vec = pl.BlockSpec((D,), lambda i: (0,))
_spec_w = pl.BlockSpec((D, D), lambda i: (0, 0))

_epi0 = pl.pallas_call(
    _epi0_body,
    grid=(N // BT,),
    in_specs=[_spec_aggp, _spec_dinv, _spec_vec, _spec_vec, _spec_vec,
              _spec_w],
    out_specs=(_spec_rows, _spec_rows),
    out_shape=(jax.ShapeDtypeStruct((N, D), jnp.float32),
               jax.ShapeDtypeStruct((N, D), jnp.float32)),
)

_epi_res = pl.pallas_call(
    _epi_res_body,
    grid=(N // BT,),
    in_specs=[_spec_aggp, _spec_dinv, _spec_vec, _spec_vec, _spec_vec,
              _spec_w, _spec_rows],
    out_specs=(_spec_rows, _spec_rows),
    out_shape=(jax.ShapeDtypeStruct((N, D), jnp.float32),
               jax.ShapeDtypeStruct((N, D), jnp.float32)),
)

_fin = pl.pallas_call(
    _fin_body,
    grid=(N // BT,),
    in_specs=[_spec_aggp, _spec_dinv, _spec_vec, _spec_vec, _spec_vec,
              _spec_rows],
    out_specs=_spec_rows,
    out_shape=jax.ShapeDtypeStruct((N, D), jnp.float32),
)


def kernel(emb, W0, b0, gamma0, beta0, W1, b1, gamma1, beta1,
           W2, b2, gamma2, beta2, edge_index):
    row = edge_index[0]
    col = edge_index[1]
    znd = jnp.zeros((N, D), jnp.float32)
    z1 = jnp.zeros((N,), jnp.float32)

    degp = _sc_degree(col, z1)
    dinv = jnp.reshape(_dinv(degp), (N, 1))

    hs = _mm_scale(emb, W0, dinv)
    aggp = _sc_scatter(row, col, hs, znd)
    x1, hs = _epi0(aggp, dinv, b0, gamma0, beta0, W1)
    aggp = _sc_scatter(row, col, hs, znd)
    x2, hs = _epi_res(aggp, dinv, b1, gamma1, beta1, W2, x1)
    aggp = _sc_scatter(row, col, hs, znd)
    return _fin(aggp, dinv, b2, gamma2, beta2, x2)
